# Initial kernel scaffold; baseline (speedup 1.0000x reference)
import jax
import jax.numpy as jnp
from jax.experimental import pallas as pl


def kernel(weight, seq_length):
    raise NotImplementedError("write your pallas kernel here")


# speedup vs baseline: 22.5831x; 22.5831x over previous
import functools
import math

import jax
import jax.numpy as jnp
from jax import lax
from jax.experimental import pallas as pl
from jax.experimental.pallas import tpu as pltpu
from jax.experimental.pallas import tpu_sc as plsc

S = 2048
H = 32
NB2 = 16
MAX_EXACT = 8
TBL = 2 * S - 1
TBL_PAD = 4112
NPH = 8
L = 16
NFLIGHT = 8

_mesh = plsc.VectorSubcoreMesh(core_axis_name="c", subcore_axis_name="s")


@functools.partial(
    pl.kernel,
    mesh=_mesh,
    compiler_params=pltpu.CompilerParams(
        needs_layout_passes=False,
        use_tc_tiling_on_sc=False,
    ),
    out_type=jax.ShapeDtypeStruct((1, H, S, S), jnp.float32),
    scratch_types=[
        pltpu.VMEM((H * H,), jnp.float32),
        pltpu.VMEM((TBL_PAD,), jnp.int32),
        pltpu.VMEM((TBL_PAD,), jnp.float32),
        pltpu.VMEM((NPH * 4096,), jnp.float32),
        pltpu.SemaphoreType.DMA,
    ],
)
def _bias_sc(weight_hbm, bucket_hbm, out_hbm, w_v, b_v, vh_v, ph_v, sem):
    h = lax.axis_index("s") * 2 + lax.axis_index("c")

    pltpu.sync_copy(weight_hbm, w_v)
    pltpu.sync_copy(bucket_hbm, b_v)

    hvec = jnp.full((L,), h, dtype=jnp.int32)

    def gather_body(k, carry):
        idx = b_v[pl.ds(k * L, L)] * H + hvec
        vh_v[pl.ds(k * L, L)] = plsc.load_gather(w_v, [idx])
        return carry

    lax.fori_loop(0, TBL_PAD // L, gather_body, 0)

    for p in range(NPH):
        def phase_body(k, carry, p=p):
            ph_v[pl.ds(p * 4096 + k * L, L)] = vh_v[pl.ds(k * L + p, L)]
            return carry

        lax.fori_loop(0, 4096 // L, phase_body, 0)

    def wait_one_row():
        pltpu.make_async_copy(
            ph_v.at[pl.ds(0, S)], out_hbm.at[0, h, 0], sem
        ).wait()

    for p in range(NPH):
        def row_body(m, carry, p=p):
            i = (NPH - 1 - p) + NPH * m
            q = pl.multiple_of(p * 4096 + 2040 - NPH * m, NPH)
            pltpu.async_copy(ph_v.at[pl.ds(q, S)], out_hbm.at[0, h, i], sem)

            @pl.when(p * (S // NPH) + m >= NFLIGHT)
            def _():
                wait_one_row()

            return carry

        lax.fori_loop(0, S // NPH, row_body, 0)

    for _ in range(NFLIGHT):
        wait_one_row()


def kernel(weight, seq_length):
    del seq_length
    d = jnp.arange(-(S - 1), S, dtype=jnp.int32)
    rel_buckets = (d > 0).astype(jnp.int32) * NB2
    ad = jnp.abs(d)
    is_small = ad < MAX_EXACT
    rp_safe = jnp.maximum(ad, 1)
    large = MAX_EXACT + (
        jnp.log(rp_safe.astype(jnp.float32) / MAX_EXACT)
        / math.log(128 / MAX_EXACT)
        * (NB2 - MAX_EXACT)
    ).astype(jnp.int32)
    large = jnp.minimum(large, NB2 - 1)
    buckets = rel_buckets + jnp.where(is_small, ad, large)
    buckets = jnp.pad(buckets, (0, TBL_PAD - TBL))
    return _bias_sc(weight.reshape(-1), buckets)
